# baseline (speedup 1.0000x reference)
import functools

import jax
import jax.numpy as jnp
from jax import lax
from jax.experimental import pallas as pl
from jax.experimental.pallas import tpu as pltpu
from jax.experimental.pallas import tpu_sc as plsc

NX = 64
BATCH = 16384
NUM_CORES = 2
B_PER_C = BATCH // NUM_CORES
CHS = 512
NCH = B_PER_C // CHS


@functools.partial(
    pl.kernel,
    out_type=jax.ShapeDtypeStruct((BATCH, NX), jnp.float32),
    mesh=plsc.ScalarSubcoreMesh(axis_name="c", num_cores=NUM_CORES),
    scratch_types=[
        pltpu.SMEM((CHS,), jnp.int32),
        pltpu.VMEM_SHARED((B_PER_C, NX), jnp.float32),
        pltpu.SemaphoreType.DMA,
    ],
    compiler_params=pltpu.CompilerParams(use_tc_tiling_on_sc=True),
)
def _sc_gather(ids_hbm, table_hbm, out_hbm, ids_s, rows_sh, sem):
    cid = lax.axis_index("c")
    base = cid * B_PER_C

    def chunk(cn, carry):
        pltpu.sync_copy(ids_hbm.at[pl.ds(base + cn * CHS, CHS)], ids_s)

        def issue(j, icarry):
            r = ids_s[j]
            pltpu.make_async_copy(
                table_hbm.at[r], rows_sh.at[cn * CHS + j], sem
            ).start()
            return icarry

        lax.fori_loop(0, CHS, issue, 0)
        return carry

    lax.fori_loop(0, NCH, chunk, 0)

    def drain(j, carry):
        pltpu.make_async_copy(table_hbm.at[0], rows_sh.at[j], sem).wait()
        return carry

    lax.fori_loop(0, B_PER_C, drain, 0)

    pltpu.sync_copy(rows_sh, out_hbm.at[pl.ds(base, B_PER_C)])


def kernel(ids, start_state):
    return _sc_gather(ids.astype(jnp.int32), start_state)
